# baseline (speedup 1.0000x reference)
import functools

import jax
import jax.numpy as jnp
from jax import lax
from jax.experimental import pallas as pl
from jax.experimental.pallas import tpu as pltpu
from jax.experimental.pallas import tpu_sc as plsc

B = 16384
D = 32
NC = 2
NS = 16
NW = NC * NS
BPW = B // NW
CH = 128
NCH = BPW // CH
NG = CH // 16

_sc_mesh = plsc.VectorSubcoreMesh(core_axis_name="c", subcore_axis_name="s")


@functools.partial(
    pl.kernel,
    mesh=_sc_mesh,
    compiler_params=pltpu.CompilerParams(use_tc_tiling_on_sc=True,
                                         needs_layout_passes=False),
    out_type=jax.ShapeDtypeStruct((B, 128), jnp.float32),
    scratch_types=[
        pltpu.VMEM((NCH, CH), jnp.int32),
        pltpu.VMEM((NCH, CH), jnp.int32),
        pltpu.VMEM((NCH, CH), jnp.int32),
        pltpu.VMEM((NCH, CH), jnp.int32),
        pltpu.VMEM((CH, 128), jnp.float32),
        pltpu.VMEM((BPW, 128), jnp.float32),
        pltpu.SemaphoreType.DMA,
    ],
)
def _sc_gather(uids, mids, gue, gme, mue, mme, out_h,
               cu_v, cm_v, su_v, sm_v, buf_v, pack_v, sem):
    wid = lax.axis_index("s") * NC + lax.axis_index("c")
    base = wid * BPW
    for c in range(NCH):
        pltpu.sync_copy(uids.at[pl.ds(base + c * CH, CH)], cu_v.at[c])
        pltpu.sync_copy(mids.at[pl.ds(base + c * CH, CH)], cm_v.at[c])
    for c in range(NCH):
        for k in range(CH // 16):
            sl = pl.ds(k * 16, 16)
            rawu = cu_v[c, sl]
            rawm = cm_v[c, sl]
            cu_v[c, sl] = rawu >> 2
            su_v[c, sl] = (rawu & 3) * 32
            cm_v[c, sl] = rawm >> 2
            sm_v[c, sl] = (rawm & 3) * 32

    iota = lax.iota(jnp.int32, 16)

    def select(c, sref, colbase):
        def body(g, _):
            rowv = g * 16 + iota
            offv = sref[c, pl.ds(g * 16, 16)]
            prow = c * CH + rowv
            for k in range(D):
                v = plsc.load_gather(buf_v, [rowv, offv + k])
                plsc.store_scatter(pack_v, [prow, jnp.full((16,), colbase + k, jnp.int32)], v)
            return 0
        lax.fori_loop(0, NG, body, 0, unroll=False)

    def chunk(c, _):
        cp = pltpu.async_copy(gue.at[cu_v.at[c]], buf_v, sem)
        cp.wait()
        select(c, su_v, 0)
        cp = pltpu.async_copy(gme.at[cm_v.at[c]], buf_v, sem)
        cp.wait()
        select(c, sm_v, 32)
        cp = pltpu.async_copy(mue.at[cu_v.at[c]], buf_v, sem)
        cp.wait()
        select(c, su_v, 64)
        cp = pltpu.async_copy(mme.at[cm_v.at[c]], buf_v, sem)
        cp.wait()
        select(c, sm_v, 96)
        return 0

    lax.fori_loop(0, NCH, chunk, 0, unroll=False)
    pltpu.sync_copy(pack_v, out_h.at[pl.ds(base, BPW)])


BLK = 2048


def _tc_body(x, w1a, w1b, b1, w2, b2, wg, wh, bo, out_ref):
    f32 = jnp.float32
    gu = x[:, 0:D]
    gm = x[:, D:2 * D]
    mu = x[:, 2 * D:3 * D]
    mm = x[:, 3 * D:4 * D]
    h1 = jnp.dot(mu, w1a[...], preferred_element_type=f32)
    h1 = h1 + jnp.dot(mm, w1b[...], preferred_element_type=f32)
    h1 = jnp.maximum(h1 + b1[...], 0.0)
    h2 = jnp.maximum(jnp.dot(h1, w2[...], preferred_element_type=f32) + b2[...], 0.0)
    logit = jnp.dot(gu * gm, wg[...], preferred_element_type=f32)
    logit = logit + jnp.dot(h2, wh[...], preferred_element_type=f32)
    logit = logit + bo[...]
    out_ref[...] = 1.0 / (1.0 + jnp.exp(-logit))


def _tc_mlp(x, w1a, w1b, b1, W2, b2, wg, wh, bout):
    grid = B // BLK
    blk2 = lambda shape: pl.BlockSpec(shape, lambda i: (0, 0))
    blk1 = lambda shape: pl.BlockSpec(shape, lambda i: (0,))
    return pl.pallas_call(
        _tc_body,
        grid=(grid,),
        in_specs=[
            pl.BlockSpec((BLK, 128), lambda i: (i, 0)),
            blk2(w1a.shape), blk2(w1b.shape), blk1(b1.shape),
            blk2(W2.shape), blk1(b2.shape),
            blk2(wg.shape), blk2(wh.shape), blk1(bout.shape),
        ],
        out_specs=pl.BlockSpec((BLK, 1), lambda i: (i, 0)),
        out_shape=jax.ShapeDtypeStruct((B, 1), jnp.float32),
    )(x, w1a, w1b, b1, W2, b2, wg, wh, bout)


def kernel(user_ids, movie_ids, gmf_user_emb, gmf_movie_emb,
           mlp_user_emb, mlp_movie_emb, W1, b1, W2, b2, Wout, bout):
    NU = gmf_user_emb.shape[0]
    NM = gmf_movie_emb.shape[0]
    x = _sc_gather(user_ids, movie_ids,
                   gmf_user_emb.reshape(NU // 4, 128),
                   gmf_movie_emb.reshape(NM // 4, 128),
                   mlp_user_emb.reshape(NU // 4, 128),
                   mlp_movie_emb.reshape(NM // 4, 128))
    out = _tc_mlp(x, W1[:D], W1[D:], b1, W2, b2, Wout[:D], Wout[D:], bout)
    return out[:, 0]


# speedup vs baseline: 1.5771x; 1.5771x over previous
import functools

import jax
import jax.numpy as jnp
from jax import lax
from jax.experimental import pallas as pl
from jax.experimental.pallas import tpu as pltpu
from jax.experimental.pallas import tpu_sc as plsc

B = 16384
D = 32
NC = 2
NS = 16
NW = NC * NS
BPW = B // NW
NG = BPW // 16

_sc_mesh = plsc.VectorSubcoreMesh(core_axis_name="c", subcore_axis_name="s")


@functools.partial(
    pl.kernel,
    mesh=_sc_mesh,
    compiler_params=pltpu.CompilerParams(use_tc_tiling_on_sc=True),
    out_type=jax.ShapeDtypeStruct((B, 128), jnp.float32),
    scratch_types=[
        pltpu.VMEM((BPW,), jnp.int32),
        pltpu.VMEM((BPW,), jnp.int32),
        pltpu.VMEM((BPW, 128), jnp.float32),
        pltpu.SemaphoreType.DMA,
    ],
)
def _sc_gather(uids, mids, gue, gme, mue, mme, out_h,
               uidx_v, midx_v, pack_v, sem):
    wid = lax.axis_index("s") * NC + lax.axis_index("c")
    base = wid * BPW
    pltpu.sync_copy(uids.at[pl.ds(base, BPW)], uidx_v)
    pltpu.sync_copy(mids.at[pl.ds(base, BPW)], midx_v)

    def body(g, _):
        uvec = uidx_v[pl.ds(g * 16, 16)]
        mvec = midx_v[pl.ds(g * 16, 16)]
        for i in range(16):
            j = g * 16 + i
            ru = uvec[i]
            rm = mvec[i]
            pltpu.async_copy(gue.at[ru], pack_v.at[j, pl.ds(0, D)], sem)
            pltpu.async_copy(gme.at[rm], pack_v.at[j, pl.ds(D, D)], sem)
            pltpu.async_copy(mue.at[ru], pack_v.at[j, pl.ds(2 * D, D)], sem)
            pltpu.async_copy(mme.at[rm], pack_v.at[j, pl.ds(3 * D, D)], sem)
        return 0

    lax.fori_loop(0, NG, body, 0, unroll=False)
    pltpu.make_async_copy(out_h.at[pl.ds(base, BPW)], pack_v, sem).wait()
    pltpu.sync_copy(pack_v, out_h.at[pl.ds(base, BPW)])


BLK = 2048


def _tc_body(x, w1a, w1b, b1, w2, b2, wg, wh, bo, out_ref):
    f32 = jnp.float32
    gu = x[:, 0:D]
    gm = x[:, D:2 * D]
    mu = x[:, 2 * D:3 * D]
    mm = x[:, 3 * D:4 * D]
    h1 = jnp.dot(mu, w1a[...], preferred_element_type=f32)
    h1 = h1 + jnp.dot(mm, w1b[...], preferred_element_type=f32)
    h1 = jnp.maximum(h1 + b1[...], 0.0)
    h2 = jnp.maximum(jnp.dot(h1, w2[...], preferred_element_type=f32) + b2[...], 0.0)
    logit = jnp.dot(gu * gm, wg[...], preferred_element_type=f32)
    logit = logit + jnp.dot(h2, wh[...], preferred_element_type=f32)
    logit = logit + bo[...]
    out_ref[...] = 1.0 / (1.0 + jnp.exp(-logit))


def _tc_mlp(x, w1a, w1b, b1, W2, b2, wg, wh, bout):
    grid = B // BLK
    blk2 = lambda shape: pl.BlockSpec(shape, lambda i: (0, 0))
    blk1 = lambda shape: pl.BlockSpec(shape, lambda i: (0,))
    return pl.pallas_call(
        _tc_body,
        grid=(grid,),
        in_specs=[
            pl.BlockSpec((BLK, 128), lambda i: (i, 0)),
            blk2(w1a.shape), blk2(w1b.shape), blk1(b1.shape),
            blk2(W2.shape), blk1(b2.shape),
            blk2(wg.shape), blk2(wh.shape), blk1(bout.shape),
        ],
        out_specs=pl.BlockSpec((BLK, 1), lambda i: (i, 0)),
        out_shape=jax.ShapeDtypeStruct((B, 1), jnp.float32),
    )(x, w1a, w1b, b1, W2, b2, wg, wh, bout)


def kernel(user_ids, movie_ids, gmf_user_emb, gmf_movie_emb,
           mlp_user_emb, mlp_movie_emb, W1, b1, W2, b2, Wout, bout):
    x = _sc_gather(user_ids, movie_ids, gmf_user_emb, gmf_movie_emb,
                   mlp_user_emb, mlp_movie_emb)
    out = _tc_mlp(x, W1[:D], W1[D:], b1, W2, b2, Wout[:D], Wout[D:], bout)
    return out[:, 0]
